# baseline (speedup 1.0000x reference)
import functools

import jax
import jax.numpy as jnp
from jax import lax
from jax.experimental import pallas as pl
from jax.experimental.pallas import tpu as pltpu
from jax.experimental.pallas import tpu_sc as plsc

N = 10000
E = 320000
D = 128
DH = D // 2

NC = 2
NS = 16

EPS = E // NS
CHUNK = 128
NCHUNK = 162
EPAD = NCHUNK * CHUNK - EPS
PADROWS = 80
NP = N + PADROWS
NSTRIPE_R = 80
NZSTRIPE = NP // NSTRIPE_R
NSTRIPE = N // NSTRIPE_R


def _sc_body(feat_hbm, src_hbm, out_hbm,
             sidx_v, didx_v, rows_v, agg_s, gsem, ssem):
    c = lax.axis_index("c")
    s = lax.axis_index("s")

    zbuf = rows_v.at[0, pl.ds(0, NSTRIPE_R)]

    def _zstore(i, carry):
        rows_v[0, i // 2, pl.ds((i % 2) * 32, 32)] = (
            jnp.zeros((32,), jnp.bfloat16))
        return carry
    lax.fori_loop(0, NSTRIPE_R * (DH // 32), _zstore, 0)

    def _zcopy(t, carry):
        idx = s + NS * t

        @pl.when(idx < NZSTRIPE)
        def _():
            pltpu.sync_copy(zbuf, agg_s.at[pl.ds(idx * NSTRIPE_R, NSTRIPE_R)])
        return carry
    lax.fori_loop(0, pl.cdiv(NZSTRIPE, NS), _zcopy, 0)
    plsc.subcore_barrier()

    pltpu.sync_copy(src_hbm.at[0, s], sidx_v)
    pltpu.sync_copy(src_hbm.at[1, s], didx_v)

    for u in range(3):
        pltpu.async_copy(feat_hbm.at[c].at[sidx_v.at[u]], rows_v.at[u], gsem.at[u])

    def _step(t, carry):
        base = (t % 2) * 3
        nbase = 3 - base
        scatters = []
        for u in range(3):
            j = 3 * t + u

            @pl.when(j + 3 < NCHUNK)
            def _():
                pltpu.async_copy(feat_hbm.at[c].at[sidx_v.at[j + 3]],
                                 rows_v.at[nbase + u], gsem.at[nbase + u])
            pltpu.make_async_copy(feat_hbm.at[c].at[sidx_v.at[j]],
                                  rows_v.at[base + u], gsem.at[base + u]).wait()
            scatters.append(pltpu.async_copy(
                rows_v.at[base + u], agg_s.at[didx_v.at[j]],
                ssem.at[base + u], add=True))
        for d in scatters:
            d.wait()
        return carry
    lax.fori_loop(0, NCHUNK // 3, _step, 0)
    plsc.subcore_barrier()

    def _ocopy(t, carry):
        idx = s + NS * t

        @pl.when(idx < NSTRIPE)
        def _():
            pltpu.sync_copy(agg_s.at[pl.ds(idx * NSTRIPE_R, NSTRIPE_R)],
                            out_hbm.at[c, pl.ds(idx * NSTRIPE_R, NSTRIPE_R)])
        return carry
    lax.fori_loop(0, pl.cdiv(NSTRIPE, NS), _ocopy, 0)


@functools.cache
def _sc_agg():
    mesh = plsc.VectorSubcoreMesh(
        core_axis_name="c", subcore_axis_name="s",
        num_cores=NC, num_subcores=NS)
    return pl.kernel(
        _sc_body,
        out_type=jax.ShapeDtypeStruct((NC, N, DH), jnp.bfloat16),
        mesh=mesh,
        scratch_types=[
            pltpu.VMEM((NCHUNK, CHUNK), jnp.int32),
            pltpu.VMEM((NCHUNK, CHUNK), jnp.int32),
            pltpu.VMEM((6, CHUNK, DH), jnp.bfloat16),
            pltpu.VMEM_SHARED((NP, DH), jnp.bfloat16),
            pltpu.SemaphoreType.DMA((6,)),
            pltpu.SemaphoreType.DMA((6,)),
        ],
        compiler_params=pltpu.CompilerParams(use_tc_tiling_on_sc=False),
    )


BR = 1000


def _tc_body(f_ref, a0_ref, a1_ref, dn_ref, w0t_ref, w1t_ref,
             b0_ref, b1_ref, o_ref):
    h0 = jnp.dot(f_ref[...], w0t_ref[...],
                 preferred_element_type=jnp.float32) + b0_ref[...]
    agg = (jnp.concatenate([a0_ref[0], a1_ref[0]], axis=1)
           .astype(jnp.float32) * dn_ref[...])
    h1 = jnp.dot(agg, w1t_ref[...],
                 preferred_element_type=jnp.float32) + b1_ref[...]
    o_ref[:, :D] = h0
    o_ref[:, D:] = h1


_tc_fuse = pl.pallas_call(
    _tc_body,
    grid=(N // BR,),
    in_specs=[
        pl.BlockSpec((BR, D), lambda i: (i, 0)),
        pl.BlockSpec((1, BR, DH), lambda i: (0, i, 0)),
        pl.BlockSpec((1, BR, DH), lambda i: (1, i, 0)),
        pl.BlockSpec((BR, 1), lambda i: (i, 0)),
        pl.BlockSpec((D, D), lambda i: (0, 0)),
        pl.BlockSpec((D, D), lambda i: (0, 0)),
        pl.BlockSpec((1, D), lambda i: (0, 0)),
        pl.BlockSpec((1, D), lambda i: (0, 0)),
    ],
    out_specs=pl.BlockSpec((BR, 2 * D), lambda i: (i, 0)),
    out_shape=jax.ShapeDtypeStruct((N, 2 * D), jnp.float32),
)


def kernel(features, edge_index, D_norm, W0, b0, W1, b1):
    featc = features.reshape(N, NC, DH).transpose(1, 0, 2).astype(jnp.bfloat16)
    lane = jnp.arange(EPAD, dtype=jnp.int32)[None, None, :]
    sub = jnp.arange(NS, dtype=jnp.int32)[None, :, None]
    pad_src = (sub * 1249 + lane * 257) % N
    pad_dst = N + (sub * 5 + lane) % PADROWS
    pad = jnp.concatenate([pad_src, pad_dst], axis=0)
    idx2 = jnp.concatenate([edge_index.reshape(2, NS, EPS), pad], axis=2)
    idx4 = idx2.reshape(2, NS, NCHUNK, CHUNK)
    agg = _sc_agg()(featc, idx4)
    return _tc_fuse(features, agg, agg, D_norm,
                    W0.T, W1.T, b0.reshape(1, D), b1.reshape(1, D))


# speedup vs baseline: 1.0484x; 1.0484x over previous
import functools

import jax
import jax.numpy as jnp
from jax import lax
from jax.experimental import pallas as pl
from jax.experimental.pallas import tpu as pltpu
from jax.experimental.pallas import tpu_sc as plsc

N = 10000
E = 320000
D = 128
DH = D // 2

NC = 2
NS = 16

EPS = E // (NC * NS)
CHUNK = 128
NCHUNK = 84
EPAD = NCHUNK * CHUNK - EPS
PADROWS = 80
NP = N + PADROWS
NSTRIPE_R = 80
NZSTRIPE = NP // NSTRIPE_R
NSTRIPE = N // NSTRIPE_R


def _sc_body(feat_hbm, src_hbm, out_hbm,
             sidx_v, didx_v, rows_v, agg_s, gsem, ssem):
    c = lax.axis_index("c")
    s = lax.axis_index("s")

    zbuf = rows_v.at[0, pl.ds(0, NSTRIPE_R)]

    def _zstore(i, carry):
        rows_v[0, i // 4, pl.ds((i % 4) * 32, 32)] = (
            jnp.zeros((32,), jnp.bfloat16))
        return carry
    lax.fori_loop(0, NSTRIPE_R * (D // 32), _zstore, 0)

    def _zcopy(t, carry):
        idx = s + NS * t

        @pl.when(idx < NZSTRIPE)
        def _():
            pltpu.sync_copy(zbuf, agg_s.at[pl.ds(idx * NSTRIPE_R, NSTRIPE_R)])
        return carry
    lax.fori_loop(0, pl.cdiv(NZSTRIPE, NS), _zcopy, 0)
    plsc.subcore_barrier()

    pltpu.sync_copy(src_hbm.at[0, c, s], sidx_v)
    pltpu.sync_copy(src_hbm.at[1, c, s], didx_v)

    for u in range(3):
        pltpu.async_copy(feat_hbm.at[sidx_v.at[u]], rows_v.at[u], gsem.at[u])

    def _step(t, carry):
        base = (t % 2) * 3
        nbase = 3 - base
        scatters = []
        for u in range(3):
            j = 3 * t + u

            @pl.when(j + 3 < NCHUNK)
            def _():
                pltpu.async_copy(feat_hbm.at[sidx_v.at[j + 3]],
                                 rows_v.at[nbase + u], gsem.at[nbase + u])
            pltpu.make_async_copy(feat_hbm.at[sidx_v.at[j]],
                                  rows_v.at[base + u], gsem.at[base + u]).wait()
            scatters.append(pltpu.async_copy(
                rows_v.at[base + u], agg_s.at[didx_v.at[j]],
                ssem.at[base + u], add=True))
        for d in scatters:
            d.wait()
        return carry
    lax.fori_loop(0, NCHUNK // 3, _step, 0)
    plsc.subcore_barrier()

    def _ocopy(t, carry):
        idx = s + NS * t

        @pl.when(idx < NSTRIPE)
        def _():
            pltpu.sync_copy(agg_s.at[pl.ds(idx * NSTRIPE_R, NSTRIPE_R)],
                            out_hbm.at[c, pl.ds(idx * NSTRIPE_R, NSTRIPE_R)])
        return carry
    lax.fori_loop(0, pl.cdiv(NSTRIPE, NS), _ocopy, 0)


@functools.cache
def _sc_agg():
    mesh = plsc.VectorSubcoreMesh(
        core_axis_name="c", subcore_axis_name="s",
        num_cores=NC, num_subcores=NS)
    return pl.kernel(
        _sc_body,
        out_type=jax.ShapeDtypeStruct((NC, N, D), jnp.bfloat16),
        mesh=mesh,
        scratch_types=[
            pltpu.VMEM((NCHUNK, CHUNK), jnp.int32),
            pltpu.VMEM((NCHUNK, CHUNK), jnp.int32),
            pltpu.VMEM((6, CHUNK, D), jnp.bfloat16),
            pltpu.VMEM_SHARED((NP, D), jnp.bfloat16),
            pltpu.SemaphoreType.DMA((6,)),
            pltpu.SemaphoreType.DMA((6,)),
        ],
        compiler_params=pltpu.CompilerParams(use_tc_tiling_on_sc=False),
    )


BR = 1000


def _tc_body(f_ref, a0_ref, a1_ref, dn_ref, w0t_ref, w1t_ref,
             b0_ref, b1_ref, o_ref):
    h0 = jnp.dot(f_ref[...], w0t_ref[...],
                 preferred_element_type=jnp.float32) + b0_ref[...]
    agg = ((a0_ref[0].astype(jnp.float32) + a1_ref[0].astype(jnp.float32))
           * dn_ref[...])
    h1 = jnp.dot(agg, w1t_ref[...],
                 preferred_element_type=jnp.float32) + b1_ref[...]
    o_ref[:, :D] = h0
    o_ref[:, D:] = h1


_tc_fuse = pl.pallas_call(
    _tc_body,
    grid=(N // BR,),
    in_specs=[
        pl.BlockSpec((BR, D), lambda i: (i, 0)),
        pl.BlockSpec((1, BR, D), lambda i: (0, i, 0)),
        pl.BlockSpec((1, BR, D), lambda i: (1, i, 0)),
        pl.BlockSpec((BR, 1), lambda i: (i, 0)),
        pl.BlockSpec((D, D), lambda i: (0, 0)),
        pl.BlockSpec((D, D), lambda i: (0, 0)),
        pl.BlockSpec((1, D), lambda i: (0, 0)),
        pl.BlockSpec((1, D), lambda i: (0, 0)),
    ],
    out_specs=pl.BlockSpec((BR, 2 * D), lambda i: (i, 0)),
    out_shape=jax.ShapeDtypeStruct((N, 2 * D), jnp.float32),
)


def kernel(features, edge_index, D_norm, W0, b0, W1, b1):
    featc = features.astype(jnp.bfloat16)
    lane = jnp.arange(EPAD, dtype=jnp.int32)[None, None, None, :]
    sub = jnp.arange(NS, dtype=jnp.int32)[None, None, :, None]
    pad_src = jnp.broadcast_to((sub * 1249 + lane * 257) % N,
                               (1, NC, NS, EPAD))
    pad_dst = jnp.broadcast_to(N + (sub * 5 + lane) % PADROWS,
                               (1, NC, NS, EPAD))
    pad = jnp.concatenate([pad_src, pad_dst], axis=0)
    idx2 = jnp.concatenate([edge_index.reshape(2, NC, NS, EPS), pad], axis=3)
    idx4 = idx2.reshape(2, NC, NS, NCHUNK, CHUNK)
    agg = _sc_agg()(featc, idx4)
    return _tc_fuse(features, agg, agg, D_norm,
                    W0.T, W1.T, b0.reshape(1, D), b1.reshape(1, D))


# speedup vs baseline: 1.0495x; 1.0011x over previous
import functools

import jax
import jax.numpy as jnp
from jax import lax
from jax.experimental import pallas as pl
from jax.experimental.pallas import tpu as pltpu
from jax.experimental.pallas import tpu_sc as plsc

N = 10000
E = 320000
D = 128

NC = 2
NS = 16

EPS = E // (NC * NS)
CHUNK = 128
NCHUNK = 84
EPAD = NCHUNK * CHUNK - EPS
PADROWS = 80
NP = N + PADROWS
NSTRIPE_R = 80
NZSTRIPE = NP // NSTRIPE_R
NSTRIPE = N // NSTRIPE_R


def _sc_body(feat_hbm, src_hbm, out_hbm,
             sidx_v, didx_v, rows_v, agg_s, gsem, ssem):
    c = lax.axis_index("c")
    s = lax.axis_index("s")

    zbuf = rows_v.at[0, pl.ds(0, NSTRIPE_R)]

    def _zstore(i, carry):
        rows_v[0, i // 4, pl.ds((i % 4) * 32, 32)] = (
            jnp.zeros((32,), jnp.bfloat16))
        return carry
    lax.fori_loop(0, NSTRIPE_R * (D // 32), _zstore, 0)

    def _zcopy(t, carry):
        idx = s + NS * t

        @pl.when(idx < NZSTRIPE)
        def _():
            pltpu.sync_copy(zbuf, agg_s.at[pl.ds(idx * NSTRIPE_R, NSTRIPE_R)])
        return carry
    lax.fori_loop(0, pl.cdiv(NZSTRIPE, NS), _zcopy, 0)
    plsc.subcore_barrier()

    pltpu.sync_copy(src_hbm.at[0, c, s], sidx_v)
    pltpu.sync_copy(src_hbm.at[1, c, s], didx_v)

    for u in range(3):
        pltpu.async_copy(feat_hbm.at[sidx_v.at[u]], rows_v.at[u], gsem.at[u])

    def _step(t, carry):
        base = (t % 2) * 3
        nbase = 3 - base
        scatters = []
        for u in range(3):
            j = 3 * t + u

            @pl.when(j + 3 < NCHUNK)
            def _():
                pltpu.async_copy(feat_hbm.at[sidx_v.at[j + 3]],
                                 rows_v.at[nbase + u], gsem.at[nbase + u])
            pltpu.make_async_copy(feat_hbm.at[sidx_v.at[j]],
                                  rows_v.at[base + u], gsem.at[base + u]).wait()
            scatters.append(pltpu.async_copy(
                rows_v.at[base + u], agg_s.at[didx_v.at[j]],
                ssem.at[base + u], add=True))
        for d in scatters:
            d.wait()
        return carry
    lax.fori_loop(0, NCHUNK // 3, _step, 0)
    plsc.subcore_barrier()

    def _ocopy(t, carry):
        idx = s + NS * t

        @pl.when(idx < NSTRIPE)
        def _():
            pltpu.sync_copy(agg_s.at[pl.ds(idx * NSTRIPE_R, NSTRIPE_R)],
                            out_hbm.at[c, pl.ds(idx * NSTRIPE_R, NSTRIPE_R)])
        return carry
    lax.fori_loop(0, pl.cdiv(NSTRIPE, NS), _ocopy, 0)


@functools.cache
def _sc_agg():
    mesh = plsc.VectorSubcoreMesh(
        core_axis_name="c", subcore_axis_name="s",
        num_cores=NC, num_subcores=NS)
    return pl.kernel(
        _sc_body,
        out_type=jax.ShapeDtypeStruct((NC, N, D), jnp.bfloat16),
        mesh=mesh,
        scratch_types=[
            pltpu.VMEM((NCHUNK, CHUNK), jnp.int32),
            pltpu.VMEM((NCHUNK, CHUNK), jnp.int32),
            pltpu.VMEM((6, CHUNK, D), jnp.bfloat16),
            pltpu.VMEM_SHARED((NP, D), jnp.bfloat16),
            pltpu.SemaphoreType.DMA((6,)),
            pltpu.SemaphoreType.DMA((6,)),
        ],
        compiler_params=pltpu.CompilerParams(use_tc_tiling_on_sc=False),
    )


BR = 1000


def _tc_body(f_ref, a0_ref, a1_ref, dn_ref, w0t_ref, w1t_ref,
             b0_ref, b1_ref, o_ref):
    h0 = jnp.dot(f_ref[...], w0t_ref[...],
                 preferred_element_type=jnp.float32) + b0_ref[...]
    agg = ((a0_ref[0].astype(jnp.float32) + a1_ref[0].astype(jnp.float32))
           * dn_ref[...])
    h1 = jnp.dot(agg, w1t_ref[...],
                 preferred_element_type=jnp.float32) + b1_ref[...]
    o_ref[:, :D] = h0
    o_ref[:, D:] = h1


_tc_fuse = pl.pallas_call(
    _tc_body,
    grid=(N // BR,),
    in_specs=[
        pl.BlockSpec((BR, D), lambda i: (i, 0)),
        pl.BlockSpec((1, BR, D), lambda i: (0, i, 0)),
        pl.BlockSpec((1, BR, D), lambda i: (1, i, 0)),
        pl.BlockSpec((BR, 1), lambda i: (i, 0)),
        pl.BlockSpec((D, D), lambda i: (0, 0)),
        pl.BlockSpec((D, D), lambda i: (0, 0)),
        pl.BlockSpec((1, D), lambda i: (0, 0)),
        pl.BlockSpec((1, D), lambda i: (0, 0)),
    ],
    out_specs=pl.BlockSpec((BR, 2 * D), lambda i: (i, 0)),
    out_shape=jax.ShapeDtypeStruct((N, 2 * D), jnp.float32),
)


def kernel(features, edge_index, D_norm, W0, b0, W1, b1):
    featc = features.astype(jnp.bfloat16)
    lane = jnp.arange(EPAD, dtype=jnp.int32)[None, None, None, :]
    sub = jnp.arange(NS, dtype=jnp.int32)[None, None, :, None]
    pad_src = jnp.broadcast_to((sub * 1249 + lane * 257) % N,
                               (1, NC, NS, EPAD))
    pad_dst = jnp.broadcast_to(N + (sub * 5 + lane) % PADROWS,
                               (1, NC, NS, EPAD))
    pad = jnp.concatenate([pad_src, pad_dst], axis=0)
    idx2 = jnp.concatenate([edge_index.reshape(2, NC, NS, EPS), pad], axis=3)
    idx4 = idx2.reshape(2, NC, NS, NCHUNK, CHUNK)
    agg = _sc_agg()(featc, idx4)
    return _tc_fuse(features, agg, agg, D_norm,
                    W0.T, W1.T, b0.reshape(1, D), b1.reshape(1, D))
